# Initial kernel scaffold; baseline (speedup 1.0000x reference)
import jax
import jax.numpy as jnp
from jax.experimental import pallas as pl


def kernel(x, adj, W, b):
    raise NotImplementedError("write your pallas kernel here")


# speedup vs baseline: 1.0429x; 1.0429x over previous
import jax
import jax.numpy as jnp
from jax.experimental import pallas as pl
from jax.experimental.pallas import tpu as pltpu

N, D, H = 10000, 128, 256
BM = 400


def _gcn_kernel(x_self_ref, adj_ref, x_ref, W1_ref, W2_ref, b_ref, out_ref):
    agg = jnp.dot(adj_ref[:], x_ref[:], preferred_element_type=jnp.float32)
    z = jnp.dot(x_self_ref[:], W1_ref[:], preferred_element_type=jnp.float32)
    z += jnp.dot(agg, W2_ref[:], preferred_element_type=jnp.float32)
    z += b_ref[:]
    out_ref[:] = jnp.maximum(z, 0.0)


def kernel(x, adj, W, b):
    W1 = W[:D]
    W2 = W[D:]
    b2 = b.reshape(1, H)
    grid = (N // BM,)
    return pl.pallas_call(
        _gcn_kernel,
        grid=grid,
        in_specs=[
            pl.BlockSpec((BM, D), lambda m: (m, 0)),
            pl.BlockSpec((BM, N), lambda m: (m, 0)),
            pl.BlockSpec((N, D), lambda m: (0, 0)),
            pl.BlockSpec((D, H), lambda m: (0, 0)),
            pl.BlockSpec((D, H), lambda m: (0, 0)),
            pl.BlockSpec((1, H), lambda m: (0, 0)),
        ],
        out_specs=pl.BlockSpec((BM, H), lambda m: (m, 0)),
        out_shape=jax.ShapeDtypeStruct((N, H), jnp.float32),
        compiler_params=pltpu.CompilerParams(
            dimension_semantics=("arbitrary",),
        ),
    )(x, adj, x, W1, W2, b2)
